# baseline (device time: 240057 ns/iter reference)
import jax
import jax.numpy as jnp
from jax import lax
from jax.experimental import pallas as pl
from jax.experimental.pallas import tpu as pltpu

N_DEV = 32


def kernel(x, w_mat, scale_x, scale_w):
    m_per, k = x.shape
    _, n_loc = w_mat.shape

    x8 = x.astype(jnp.float8_e4m3fn)
    wb = w_mat.astype(jnp.bfloat16)
    scale = (scale_x.astype(jnp.float32) * scale_w.astype(jnp.float32)).reshape(1, 1)

    def body(x_ref, w_ref, s_ref, out_ref, comm_ref, send_sems, recv_sems):
        my = lax.axis_index("i")
        right = lax.rem(my + 1, N_DEV)
        left = lax.rem(my + N_DEV - 1, N_DEV)

        barrier_sem = pltpu.get_barrier_semaphore()
        pl.semaphore_signal(barrier_sem, inc=1, device_id=(left,),
                            device_id_type=pl.DeviceIdType.MESH)
        pl.semaphore_signal(barrier_sem, inc=1, device_id=(right,),
                            device_id_type=pl.DeviceIdType.MESH)
        pl.semaphore_wait(barrier_sem, 2)

        s = s_ref[0]

        def compute(h):
            origin = lax.rem(my - h + 2 * N_DEV, N_DEV)
            chunk = comm_ref[h].astype(jnp.bfloat16)
            acc = jnp.dot(chunk, w_ref[...], preferred_element_type=jnp.float32)
            y = acc * s
            z = y / (1.0 + jnp.exp(-jnp.clip(y, -60.0, 60.0)))
            out_ref[pl.ds(origin * m_per, m_per), :] = z

        comm_ref[0] = x_ref[...]

        descs = []
        for h in range(1, N_DEV):
            descs.append(pltpu.make_async_remote_copy(
                src_ref=comm_ref.at[h - 1],
                dst_ref=comm_ref.at[h],
                send_sem=send_sems.at[h - 1],
                recv_sem=recv_sems.at[h - 1],
                device_id=(right,),
                device_id_type=pl.DeviceIdType.MESH,
            ))

        descs[0].start()
        compute(0)
        for h in range(1, N_DEV):
            descs[h - 1].wait_recv()
            if h < N_DEV - 1:
                descs[h].start()
            compute(h)
        for d in descs:
            d.wait_send()

    return pl.pallas_call(
        body,
        out_shape=jax.ShapeDtypeStruct((N_DEV * m_per, n_loc), jnp.float32),
        in_specs=[
            pl.BlockSpec(memory_space=pltpu.VMEM),
            pl.BlockSpec(memory_space=pltpu.VMEM),
            pl.BlockSpec(memory_space=pltpu.SMEM),
        ],
        out_specs=pl.BlockSpec(memory_space=pltpu.VMEM),
        scratch_shapes=[
            pltpu.VMEM((N_DEV, m_per, k), jnp.float8_e4m3fn),
            pltpu.SemaphoreType.DMA((N_DEV - 1,)),
            pltpu.SemaphoreType.DMA((N_DEV - 1,)),
        ],
        compiler_params=pltpu.CompilerParams(collective_id=0),
    )(x8, wb, scale.reshape(1))


# device time: 189861 ns/iter; 1.2644x vs baseline; 1.2644x over previous
import jax
import jax.numpy as jnp
from jax import lax
from jax.experimental import pallas as pl
from jax.experimental.pallas import tpu as pltpu

N_DEV = 32


def kernel(x, w_mat, scale_x, scale_w):
    m_per, k = x.shape
    _, n_loc = w_mat.shape

    x8 = x.astype(jnp.float8_e4m3fn)
    wb = w_mat.astype(jnp.bfloat16)
    scale = (scale_x.astype(jnp.float32) * scale_w.astype(jnp.float32)).reshape(1, 1)

    CW = 16
    CCW = 15

    def body(x_ref, w_ref, s_ref, out_ref, comm_ref,
             cw_ssem, cw_rsem, ccw_ssem, ccw_rsem):
        my = lax.axis_index("i")
        right = lax.rem(my + 1, N_DEV)
        left = lax.rem(my + N_DEV - 1, N_DEV)

        barrier_sem = pltpu.get_barrier_semaphore()
        pl.semaphore_signal(barrier_sem, inc=1, device_id=(left,),
                            device_id_type=pl.DeviceIdType.MESH)
        pl.semaphore_signal(barrier_sem, inc=1, device_id=(right,),
                            device_id_type=pl.DeviceIdType.MESH)
        pl.semaphore_wait(barrier_sem, 2)

        s = s_ref[0]

        def compute(slot, origin):
            chunk = comm_ref[slot].astype(jnp.bfloat16)
            acc = jnp.dot(chunk, w_ref[...], preferred_element_type=jnp.float32)
            y = acc * s
            z = y / (1.0 + jnp.exp(-jnp.clip(y, -60.0, 60.0)))
            out_ref[pl.ds(origin * m_per, m_per), :] = z

        comm_ref[0] = x_ref[...]

        cw_descs = []
        for h in range(1, CW + 1):
            cw_descs.append(pltpu.make_async_remote_copy(
                src_ref=comm_ref.at[h - 1],
                dst_ref=comm_ref.at[h],
                send_sem=cw_ssem.at[h - 1],
                recv_sem=cw_rsem.at[h - 1],
                device_id=(right,),
                device_id_type=pl.DeviceIdType.MESH,
            ))
        ccw_descs = []
        for h in range(1, CCW + 1):
            ccw_descs.append(pltpu.make_async_remote_copy(
                src_ref=comm_ref.at[0 if h == 1 else CW + h - 1],
                dst_ref=comm_ref.at[CW + h],
                send_sem=ccw_ssem.at[h - 1],
                recv_sem=ccw_rsem.at[h - 1],
                device_id=(left,),
                device_id_type=pl.DeviceIdType.MESH,
            ))

        cw_descs[0].start()
        ccw_descs[0].start()
        compute(0, my)
        for t in range(1, CW + 1):
            cw_descs[t - 1].wait_recv()
            if t < CW:
                cw_descs[t].start()
            if t <= CCW:
                ccw_descs[t - 1].wait_recv()
                if t < CCW:
                    ccw_descs[t].start()
            compute(t, lax.rem(my - t + N_DEV, N_DEV))
            if t <= CCW:
                compute(CW + t, lax.rem(my + t, N_DEV))
        for d in cw_descs:
            d.wait_send()
        for d in ccw_descs:
            d.wait_send()

    return pl.pallas_call(
        body,
        out_shape=jax.ShapeDtypeStruct((N_DEV * m_per, n_loc), jnp.float32),
        in_specs=[
            pl.BlockSpec(memory_space=pltpu.VMEM),
            pl.BlockSpec(memory_space=pltpu.VMEM),
            pl.BlockSpec(memory_space=pltpu.SMEM),
        ],
        out_specs=pl.BlockSpec(memory_space=pltpu.VMEM),
        scratch_shapes=[
            pltpu.VMEM((N_DEV, m_per, k), jnp.float8_e4m3fn),
            pltpu.SemaphoreType.DMA((CW,)),
            pltpu.SemaphoreType.DMA((CW,)),
            pltpu.SemaphoreType.DMA((CCW,)),
            pltpu.SemaphoreType.DMA((CCW,)),
        ],
        compiler_params=pltpu.CompilerParams(collective_id=0),
    )(x8, wb, scale.reshape(1))


# device time: 100459 ns/iter; 2.3896x vs baseline; 1.8899x over previous
import numpy as np
import jax
import jax.numpy as jnp
from jax import lax
from jax.experimental import pallas as pl
from jax.experimental.pallas import tpu as pltpu

N_DEV = 32
CW = 16
CCW = 15
SEG = 2


def _tables():
    logical = []
    for z in range(4):
        for (x, y) in [(0, 0), (1, 0), (1, 1), (0, 1),
                       (0, 2), (1, 2), (1, 3), (0, 3)]:
            logical.append((x, y, z))
    lidx = {c: i for i, c in enumerate(logical)}
    path = []
    for y in range(4):
        zs = range(4) if y % 2 == 0 else range(3, -1, -1)
        path.extend((y, z) for z in zs)
    cyc = [(0, y, z) for (y, z) in path] + [(1, y, z) for (y, z) in reversed(path)]
    perm = np.array([lidx[c] for c in cyc], np.int32)
    pos = np.empty(N_DEV, np.int32)
    pos[perm] = np.arange(N_DEV, dtype=np.int32)
    right = np.array([perm[(pos[l] + 1) % N_DEV] for l in range(N_DEV)], np.int32)
    left = np.array([perm[(pos[l] - 1) % N_DEV] for l in range(N_DEV)], np.int32)
    return np.stack([perm, pos, right, left])


def kernel(x, w_mat, scale_x, scale_w):
    m_per, k = x.shape
    _, n_loc = w_mat.shape
    seg_m = m_per // SEG

    x8 = x.astype(jnp.float8_e4m3fn)
    w8 = w_mat.astype(jnp.float8_e5m2)
    scale = (scale_x.astype(jnp.float32) * scale_w.astype(jnp.float32)).reshape(1)
    tab = jnp.asarray(_tables())

    def body(x_ref, w_ref, s_ref, tab_ref, out_ref, comm_ref,
             cw_ssem, cw_rsem, ccw_ssem, ccw_rsem):
        my = lax.axis_index("i")
        pos = tab_ref[1, my]
        right = tab_ref[2, my]
        left = tab_ref[3, my]

        barrier_sem = pltpu.get_barrier_semaphore()
        pl.semaphore_signal(barrier_sem, inc=1, device_id=(left,),
                            device_id_type=pl.DeviceIdType.MESH)
        pl.semaphore_signal(barrier_sem, inc=1, device_id=(right,),
                            device_id_type=pl.DeviceIdType.MESH)
        pl.semaphore_wait(barrier_sem, 2)

        s = s_ref[0]

        def silu_store(acc, origin):
            y = acc * s
            z = y / (1.0 + jnp.exp(-jnp.clip(y, -60.0, 60.0)))
            out_ref[pl.ds(origin * m_per, m_per), :] = z

        def compute(chunk_ref, origin):
            acc = lax.dot_general(chunk_ref[...], w_ref[...],
                                  (((1,), (0,)), ((), ())),
                                  preferred_element_type=jnp.float32)
            silu_store(acc, origin)

        def seg(ref, si):
            return ref.at[pl.ds(si * seg_m, seg_m), :]

        def mk(src, dst_slot, si, ssem, rsem, idx, dev):
            return pltpu.make_async_remote_copy(
                src_ref=seg(src, si),
                dst_ref=seg(comm_ref.at[dst_slot], si),
                send_sem=ssem.at[idx],
                recv_sem=rsem.at[idx],
                device_id=(dev,),
                device_id_type=pl.DeviceIdType.MESH,
            )

        cw_descs, ccw_descs = [], []
        for h in range(1, CW + 1):
            src = x_ref if h == 1 else comm_ref.at[h - 1]
            cw_descs.append([mk(src, h, si, cw_ssem, cw_rsem,
                                SEG * (h - 1) + si, right) for si in range(SEG)])
        for h in range(1, CCW + 1):
            src = x_ref if h == 1 else comm_ref.at[CW + h - 1]
            ccw_descs.append([mk(src, CW + h, si, ccw_ssem, ccw_rsem,
                                 SEG * (h - 1) + si, left) for si in range(SEG)])

        for si in range(SEG):
            cw_descs[0][si].start()
            ccw_descs[0][si].start()
        compute(x_ref, my)

        for t in range(1, CW + 1):
            for si in range(SEG):
                cw_descs[t - 1][si].wait_recv()
                if t < CW:
                    cw_descs[t][si].start()
                if t <= CCW:
                    ccw_descs[t - 1][si].wait_recv()
                    if t < CCW:
                        ccw_descs[t][si].start()
            compute(comm_ref.at[t], tab_ref[0, lax.rem(pos - t + N_DEV, N_DEV)])
            if t <= CCW:
                compute(comm_ref.at[CW + t], tab_ref[0, lax.rem(pos + t, N_DEV)])

        for ds in cw_descs + ccw_descs:
            for d in ds:
                d.wait_send()

    return pl.pallas_call(
        body,
        out_shape=jax.ShapeDtypeStruct((N_DEV * m_per, n_loc), jnp.float32),
        in_specs=[
            pl.BlockSpec(memory_space=pltpu.VMEM),
            pl.BlockSpec(memory_space=pltpu.VMEM),
            pl.BlockSpec(memory_space=pltpu.SMEM),
            pl.BlockSpec(memory_space=pltpu.SMEM),
        ],
        out_specs=pl.BlockSpec(memory_space=pltpu.VMEM),
        scratch_shapes=[
            pltpu.VMEM((N_DEV, m_per, k), jnp.float8_e4m3fn),
            pltpu.SemaphoreType.DMA((SEG * CW,)),
            pltpu.SemaphoreType.DMA((SEG * CW,)),
            pltpu.SemaphoreType.DMA((SEG * CCW,)),
            pltpu.SemaphoreType.DMA((SEG * CCW,)),
        ],
        compiler_params=pltpu.CompilerParams(collective_id=0),
    )(x8, w8, scale, tab)


# device time: 100021 ns/iter; 2.4001x vs baseline; 1.0044x over previous
import numpy as np
import jax
import jax.numpy as jnp
from jax import lax
from jax.experimental import pallas as pl
from jax.experimental.pallas import tpu as pltpu

N_DEV = 32
CW = 16
CCW = 16
SEG = 4


def _tables():
    logical = []
    for z in range(4):
        for (x, y) in [(0, 0), (1, 0), (1, 1), (0, 1),
                       (0, 2), (1, 2), (1, 3), (0, 3)]:
            logical.append((x, y, z))
    lidx = {c: i for i, c in enumerate(logical)}
    path = []
    for y in range(4):
        zs = range(4) if y % 2 == 0 else range(3, -1, -1)
        path.extend((y, z) for z in zs)
    cyc = [(0, y, z) for (y, z) in path] + [(1, y, z) for (y, z) in reversed(path)]
    perm = np.array([lidx[c] for c in cyc], np.int32)
    pos = np.empty(N_DEV, np.int32)
    pos[perm] = np.arange(N_DEV, dtype=np.int32)
    right = np.array([perm[(pos[l] + 1) % N_DEV] for l in range(N_DEV)], np.int32)
    left = np.array([perm[(pos[l] - 1) % N_DEV] for l in range(N_DEV)], np.int32)
    return np.stack([perm, pos, right, left])


def kernel(x, w_mat, scale_x, scale_w):
    m_per, k = x.shape
    _, n_loc = w_mat.shape
    seg_m = m_per // SEG

    x8 = x.astype(jnp.float8_e4m3fn)
    w8 = w_mat.astype(jnp.float8_e5m2)
    scale = (scale_x.astype(jnp.float32) * scale_w.astype(jnp.float32)).reshape(1)
    tab = jnp.asarray(_tables())

    def body(x_ref, w_ref, s_ref, tab_ref, out_ref, comm_ref,
             cw_ssem, cw_rsem, ccw_ssem, ccw_rsem):
        my = lax.axis_index("i")
        pos = tab_ref[1, my]
        right = tab_ref[2, my]
        left = tab_ref[3, my]

        barrier_sem = pltpu.get_barrier_semaphore()
        pl.semaphore_signal(barrier_sem, inc=1, device_id=(left,),
                            device_id_type=pl.DeviceIdType.MESH)
        pl.semaphore_signal(barrier_sem, inc=1, device_id=(right,),
                            device_id_type=pl.DeviceIdType.MESH)
        pl.semaphore_wait(barrier_sem, 2)

        s = s_ref[0]

        def silu_store(acc, origin):
            y = acc * s
            z = y / (1.0 + jnp.exp(-jnp.clip(y, -60.0, 60.0)))
            out_ref[pl.ds(origin * m_per, m_per), :] = z

        def compute(chunk_ref, origin):
            acc = lax.dot_general(chunk_ref[...], w_ref[...],
                                  (((1,), (0,)), ((), ())),
                                  preferred_element_type=jnp.float32)
            silu_store(acc, origin)

        def seg(ref, si):
            return ref.at[pl.ds(si * seg_m, seg_m), :]

        def cw_segs(h):
            return range(SEG) if h < CW else range(SEG // 2)

        def ccw_segs(h):
            return range(SEG) if h < CCW else range(SEG // 2, SEG)

        def mk(src, dst_slot, si, ssem, rsem, idx, dev):
            return pltpu.make_async_remote_copy(
                src_ref=seg(src, si),
                dst_ref=seg(comm_ref.at[dst_slot], si),
                send_sem=ssem.at[idx],
                recv_sem=rsem.at[idx],
                device_id=(dev,),
                device_id_type=pl.DeviceIdType.MESH,
            )

        cw_descs, ccw_descs = {}, {}
        for h in range(1, CW + 1):
            src = x_ref if h == 1 else comm_ref.at[h - 1]
            for si in cw_segs(h):
                cw_descs[h, si] = mk(src, h, si, cw_ssem, cw_rsem,
                                     SEG * (h - 1) + si, right)
        for h in range(1, CCW + 1):
            src = x_ref if h == 1 else comm_ref.at[CW + h - 1]
            dst = CW if h == CCW else CW + h
            for si in ccw_segs(h):
                ccw_descs[h, si] = mk(src, dst, si, ccw_ssem, ccw_rsem,
                                      SEG * (h - 1) + si, left)

        for si in range(SEG):
            cw_descs[1, si].start()
            ccw_descs[1, si].start()
        compute(x_ref, my)

        for t in range(1, CW + 1):
            for si in range(SEG):
                if (t, si) in cw_descs:
                    cw_descs[t, si].wait_recv()
                if (t + 1, si) in cw_descs:
                    cw_descs[t + 1, si].start()
                if (t, si) in ccw_descs:
                    ccw_descs[t, si].wait_recv()
                if (t + 1, si) in ccw_descs:
                    ccw_descs[t + 1, si].start()
            if t < CW:
                compute(comm_ref.at[t], tab_ref[0, lax.rem(pos - t + N_DEV, N_DEV)])
                compute(comm_ref.at[CW + t], tab_ref[0, lax.rem(pos + t, N_DEV)])
            else:
                compute(comm_ref.at[CW], tab_ref[0, lax.rem(pos + CW, N_DEV)])

        for d in cw_descs.values():
            d.wait_send()
        for d in ccw_descs.values():
            d.wait_send()

    return pl.pallas_call(
        body,
        out_shape=jax.ShapeDtypeStruct((N_DEV * m_per, n_loc), jnp.float32),
        in_specs=[
            pl.BlockSpec(memory_space=pltpu.VMEM),
            pl.BlockSpec(memory_space=pltpu.VMEM),
            pl.BlockSpec(memory_space=pltpu.SMEM),
            pl.BlockSpec(memory_space=pltpu.SMEM),
        ],
        out_specs=pl.BlockSpec(memory_space=pltpu.VMEM),
        scratch_shapes=[
            pltpu.VMEM((N_DEV, m_per, k), jnp.float8_e4m3fn),
            pltpu.SemaphoreType.DMA((SEG * CW,)),
            pltpu.SemaphoreType.DMA((SEG * CW,)),
            pltpu.SemaphoreType.DMA((SEG * CCW,)),
            pltpu.SemaphoreType.DMA((SEG * CCW,)),
        ],
        compiler_params=pltpu.CompilerParams(collective_id=0),
    )(x8, w8, scale, tab)
